# baseline (speedup 1.0000x reference)
import functools

import jax
import jax.numpy as jnp
from jax import lax
from jax.experimental import pallas as pl
from jax.experimental.pallas import tpu as pltpu
from jax.experimental.pallas import tpu_sc as plsc

HIDDEN = 1024
KEY_DIM = 64
MAX_MEM = 16384
T_LEN = 8
K = 8
VD = T_LEN * HIDDEN
NS = 16
L = 16
ROWS_PER_SUB = MAX_MEM // NS
GROUPS = ROWS_PER_SUB // L
GT = 8
NGT = GROUPS // GT
W_PER_SUB = KEY_DIM // NS
HCH = HIDDEN // L
KCHUNKS = 4
CHUNK_W = ROWS_PER_SUB * KEY_DIM // KCHUNKS
VCHUNK = VD // 4

_MESH = plsc.VectorSubcoreMesh(core_axis_name="c", subcore_axis_name="s")


@functools.partial(
    pl.kernel,
    out_type=jax.ShapeDtypeStruct((4 * K, VCHUNK), jnp.float32),
    mesh=_MESH,
    compiler_params=pltpu.CompilerParams(needs_layout_passes=False),
    scratch_types=[
        pltpu.VMEM((HIDDEN,), jnp.float32),
        pltpu.VMEM((W_PER_SUB * HIDDEN,), jnp.float32),
        pltpu.VMEM((ROWS_PER_SUB * KEY_DIM,), jnp.float32),
        pltpu.VMEM((L,), jnp.float32),
        pltpu.VMEM((NS * L,), jnp.float32),
        pltpu.VMEM((KEY_DIM * L,), jnp.float32),
        pltpu.VMEM((NS * L,), jnp.float32),
        pltpu.VMEM((NS * L,), jnp.int32),
        pltpu.VMEM((L,), jnp.int32),
        pltpu.VMEM((8 * L,), jnp.int32),
        pltpu.VMEM((1, VCHUNK), jnp.float32),
        pltpu.VMEM_SHARED((NS * L,), jnp.float32),
        pltpu.VMEM_SHARED((NS * L,), jnp.float32),
        pltpu.VMEM_SHARED((NS * L,), jnp.int32),
        [pltpu.SemaphoreType.DMA] * KCHUNKS,
        pltpu.SemaphoreType.DMA,
    ],
)
def _retrieve(q_hbm, keys_hbm, vals_hbm, w_hbm, out_hbm,
              q_v, w_v, keys_v, tmpf_v, qkm_v, bq_v, candv_v, candi_v,
              tmpi_v, idx8_v, row_v, qk_spmem, candv_spmem, candi_spmem,
              sem_keys, sem_row):
    cid = lax.axis_index("c")
    sid = lax.axis_index("s")
    lanes = lax.iota(jnp.int32, L)
    zero16f = jnp.zeros((L,), jnp.float32)
    neg_inf = jnp.full((L,), -jnp.inf, jnp.float32)

    pltpu.sync_copy(q_hbm, q_v)
    pltpu.sync_copy(w_hbm.at[pl.ds(sid * (W_PER_SUB * HIDDEN), W_PER_SUB * HIDDEN)],
                    w_v)
    key_cps = [
        pltpu.async_copy(
            keys_hbm.at[pl.ds(sid * (ROWS_PER_SUB * KEY_DIM) + c * CHUNK_W,
                              CHUNK_W)],
            keys_v.at[pl.ds(c * CHUNK_W, CHUNK_W)],
            sem_keys[c])
        for c in range(KCHUNKS)
    ]

    accs = [zero16f] * W_PER_SUB
    for h in range(HCH):
        qv = q_v[pl.ds(h * L, L)]
        for j in range(W_PER_SUB):
            accs[j] = accs[j] + w_v[pl.ds(j * HIDDEN + h * L, L)] * qv
    myvec = zero16f
    for j in range(W_PER_SUB):
        acc = accs[j]
        for s in (8, 4, 2, 1):
            tmpf_v[...] = acc
            acc = acc + plsc.load_gather(tmpf_v, [lanes ^ s])
        myvec = jnp.where(lanes == j, acc, myvec)
    tmpf_v[...] = myvec
    pltpu.sync_copy(tmpf_v, qk_spmem.at[pl.ds(sid * L, L)])
    plsc.subcore_barrier()
    pltpu.sync_copy(qk_spmem, qkm_v)

    def bq_body(d, _):
        qpos = (d // W_PER_SUB) * L + (d % W_PER_SUB)
        bq_v[pl.ds(d * L, L)] = plsc.load_gather(qkm_v, [lanes * 0 + qpos])
        return 0
    lax.fori_loop(0, KEY_DIM, bq_body, 0)

    colbase = lanes * KEY_DIM

    def merge_sorted_desc(sv, si, rv, ri):
        take = sv > rv
        hv = jnp.where(take, sv, rv)
        hi = jnp.where(take, si, ri)
        nrv, nri = plsc.sort_key_val(hv, hi, descending=False)
        tmpf_v[...] = nrv
        nmin = plsc.load_gather(tmpf_v, [lanes * 0])
        return nrv, nri, nmin

    def maybe_merge(vals, idxs, rv, ri, rmin):
        sv, si = plsc.sort_key_val(vals, idxs, descending=True)
        return merge_sorted_desc(sv, si, rv, ri)

    def gt_body(gt, carry):
        rv, ri, rmin = carry
        base = gt * (GT * L * KEY_DIM)
        accs = [zero16f] * GT
        def d_body(dh, accs_t):
            accs_l = list(accs_t)
            for u in range(2):
                d = dh * 2 + u
                bv = bq_v.at[pl.ds(d * L, L)][...]
                idx = colbase + d
                for gg in range(GT):
                    kv = plsc.load_gather(
                        keys_v, [idx + (base + gg * (L * KEY_DIM))])
                    accs_l[gg] = accs_l[gg] + kv * bv
            return tuple(accs_l)
        accs = lax.fori_loop(0, KEY_DIM // 2, d_body, tuple(accs))
        for gg in range(GT):
            gidx = sid * ROWS_PER_SUB + (gt * GT + gg) * L + lanes
            rv, ri, rmin = maybe_merge(accs[gg], gidx, rv, ri, rmin)
        return (rv, ri, rmin)

    carry = (neg_inf, jnp.zeros((L,), jnp.int32), neg_inf)
    gt_per_chunk = NGT // KCHUNKS
    for c in range(KCHUNKS):
        key_cps[c].wait()
        carry = lax.fori_loop(c * gt_per_chunk, (c + 1) * gt_per_chunk,
                              gt_body, carry)
    rv, ri, _ = carry

    tmpf_v[...] = rv
    tmpi_v[...] = ri
    pltpu.sync_copy(tmpf_v, candv_spmem.at[pl.ds(sid * L, L)])
    pltpu.sync_copy(tmpi_v, candi_spmem.at[pl.ds(sid * L, L)])
    plsc.subcore_barrier()
    pltpu.sync_copy(candv_spmem, candv_v)
    pltpu.sync_copy(candi_spmem, candi_v)

    def m_body(t, carry):
        mrv, mri, mmin = carry
        sv = jnp.flip(candv_v[pl.ds(t * L, L)], 0)
        si = jnp.flip(candi_v[pl.ds(t * L, L)], 0)
        return merge_sorted_desc(sv, si, mrv, mri)

    mrv, mri, _ = lax.fori_loop(
        0, NS, m_body, (neg_inf, jnp.zeros((L,), jnp.int32), neg_inf))

    best = jnp.flip(mri, 0)
    tmpi_v[...] = best
    bk = plsc.load_gather(tmpi_v, [cid * (K // 2) + lanes // W_PER_SUB])
    entry = bk * 4 + lanes % W_PER_SUB
    plsc.store_scatter(idx8_v, [lanes * 8], entry)

    pltpu.async_copy(vals_hbm.at[idx8_v.at[pl.ds(sid * 8, 1)]],
                     row_v, sem_row).wait()
    pltpu.sync_copy(row_v, out_hbm.at[pl.ds(cid * NS + sid, 1)])


def kernel(query_hidden, keys, values, W_key, top_k):
    del top_k
    out = _retrieve(query_hidden,
                    keys.reshape(-1),
                    values.reshape(4 * MAX_MEM, VCHUNK),
                    W_key.reshape(-1))
    return out.reshape(K, T_LEN, HIDDEN)


# speedup vs baseline: 8.7935x; 8.7935x over previous
import functools

import jax
import jax.numpy as jnp
from jax import lax
from jax.experimental import pallas as pl
from jax.experimental.pallas import tpu as pltpu
from jax.experimental.pallas import tpu_sc as plsc

HIDDEN = 1024
KEY_DIM = 64
MAX_MEM = 16384
T_LEN = 8
K = 8
NS = 16
L = 16
ROWS_PER_SUB = MAX_MEM // NS
GROUPS = ROWS_PER_SUB // L
GT = 8
NGT = GROUPS // GT
W_PER_SUB = KEY_DIM // NS
HCH = HIDDEN // L
KCHUNKS = 4
CROWS = ROWS_PER_SUB // KCHUNKS

_MESH = plsc.VectorSubcoreMesh(core_axis_name="c", subcore_axis_name="s")


@functools.partial(
    pl.kernel,
    out_type=jax.ShapeDtypeStruct((K, T_LEN, HIDDEN), jnp.float32),
    mesh=_MESH,
    compiler_params=pltpu.CompilerParams(needs_layout_passes=False),
    scratch_types=[
        pltpu.VMEM((HIDDEN,), jnp.float32),
        pltpu.VMEM((W_PER_SUB * HIDDEN,), jnp.float32),
        pltpu.VMEM((ROWS_PER_SUB * KEY_DIM,), jnp.float32),
        pltpu.VMEM((L,), jnp.float32),
        pltpu.VMEM((NS * L,), jnp.float32),
        pltpu.VMEM((KEY_DIM * L,), jnp.float32),
        pltpu.VMEM((NS * L,), jnp.float32),
        pltpu.VMEM((NS * L,), jnp.int32),
        pltpu.VMEM((L,), jnp.int32),
        pltpu.VMEM((8 * L,), jnp.int32),
        pltpu.VMEM((1, T_LEN, HIDDEN), jnp.float32),
        pltpu.VMEM_SHARED((NS * L,), jnp.float32),
        pltpu.VMEM_SHARED((NS * L,), jnp.float32),
        pltpu.VMEM_SHARED((NS * L,), jnp.int32),
        [pltpu.SemaphoreType.DMA] * KCHUNKS,
        pltpu.SemaphoreType.DMA,
    ],
)
def _retrieve(q_hbm, keys_hbm, vals_hbm, w_hbm, out_hbm,
              q_v, w_v, keys_v, tmpf_v, qkm_v, bq_v, candv_v, candi_v,
              tmpi_v, idx8_v, row_v, qk_spmem, candv_spmem, candi_spmem,
              sem_keys, sem_row):
    cid = lax.axis_index("c")
    sid = lax.axis_index("s")
    lanes = lax.iota(jnp.int32, L)
    zero16f = jnp.zeros((L,), jnp.float32)
    neg_inf = jnp.full((L,), -jnp.inf, jnp.float32)

    with jax.named_scope("stage_in"):
        pltpu.sync_copy(q_hbm, q_v)
        pltpu.sync_copy(w_hbm.at[pl.ds(sid * (W_PER_SUB * HIDDEN),
                                       W_PER_SUB * HIDDEN)], w_v)
        key_cps = [
            pltpu.async_copy(
                keys_hbm.at[pl.ds((sid * ROWS_PER_SUB + c * CROWS) * KEY_DIM,
                                  CROWS * KEY_DIM)],
                keys_v.at[pl.ds(c * CROWS * KEY_DIM, CROWS * KEY_DIM)],
                sem_keys[c])
            for c in range(KCHUNKS)
        ]

    with jax.named_scope("proj"):
        accs = [zero16f] * W_PER_SUB
        for h in range(HCH):
            qv = q_v[pl.ds(h * L, L)]
            for j in range(W_PER_SUB):
                accs[j] = accs[j] + w_v[pl.ds(j * HIDDEN + h * L, L)] * qv
        myvec = zero16f
        for j in range(W_PER_SUB):
            acc = accs[j]
            for s in (8, 4, 2, 1):
                tmpf_v[...] = acc
                acc = acc + plsc.load_gather(tmpf_v, [lanes ^ s])
            myvec = jnp.where(lanes == j, acc, myvec)
        tmpf_v[...] = myvec
        pltpu.sync_copy(tmpf_v, qk_spmem.at[pl.ds(sid * L, L)])
        plsc.subcore_barrier()
        pltpu.sync_copy(qk_spmem, qkm_v)

        def bq_body(d, _):
            qpos = (d // W_PER_SUB) * L + (d % W_PER_SUB)
            bq_v[pl.ds(d * L, L)] = plsc.load_gather(qkm_v, [lanes * 0 + qpos])
            return 0
        lax.fori_loop(0, KEY_DIM, bq_body, 0)

    def merge_sorted_desc(sv, si, rv, ri):
        take = sv > rv
        hv = jnp.where(take, sv, rv)
        hi = jnp.where(take, si, ri)
        nrv, nri = plsc.sort_key_val(hv, hi, descending=False)
        tmpf_v[...] = nrv
        nmin = plsc.load_gather(tmpf_v, [lanes * 0])
        return nrv, nri, nmin

    def maybe_merge(vals, idxs, rv, ri, rmin):
        sv, si = plsc.sort_key_val(vals, idxs, descending=True)
        return merge_sorted_desc(sv, si, rv, ri)

    def gt_body(gt, carry):
        rv, ri, rmin = carry
        accs = [zero16f] * GT
        def d_body(dh, accs_t):
            accs_l = list(accs_t)
            for u in range(2):
                d = dh * 2 + u
                bv = bq_v[pl.ds(d * L, L)]
                colv = lanes * KEY_DIM + d
                for gg in range(GT):
                    kv = plsc.load_gather(
                        keys_v,
                        [colv + (gt * GT + gg) * (L * KEY_DIM)])
                    accs_l[gg] = accs_l[gg] + kv * bv
            return tuple(accs_l)
        accs = lax.fori_loop(0, KEY_DIM // 2, d_body, tuple(accs))
        for gg in range(GT):
            gidx = sid * ROWS_PER_SUB + (gt * GT + gg) * L + lanes
            rv, ri, rmin = maybe_merge(accs[gg], gidx, rv, ri, rmin)
        return (rv, ri, rmin)

    with jax.named_scope("sims"):
        carry = (neg_inf, jnp.zeros((L,), jnp.int32), neg_inf)
        gt_per_chunk = NGT // KCHUNKS
        for c in range(KCHUNKS):
            key_cps[c].wait()
            carry = lax.fori_loop(c * gt_per_chunk, (c + 1) * gt_per_chunk,
                                  gt_body, carry)
        rv, ri, _ = carry

    with jax.named_scope("merge"):
        tmpf_v[...] = rv
        tmpi_v[...] = ri
        pltpu.sync_copy(tmpf_v, candv_spmem.at[pl.ds(sid * L, L)])
        pltpu.sync_copy(tmpi_v, candi_spmem.at[pl.ds(sid * L, L)])
        plsc.subcore_barrier()

    @pl.when(sid < K // 2)
    def _():
        with jax.named_scope("gather"):
            pltpu.sync_copy(candv_spmem, candv_v)
            pltpu.sync_copy(candi_spmem, candi_v)

            def m_body(t, carry):
                mrv, mri, mmin = carry
                sv = jnp.flip(candv_v[pl.ds(t * L, L)], 0)
                si = jnp.flip(candi_v[pl.ds(t * L, L)], 0)
                return merge_sorted_desc(sv, si, mrv, mri)

            mrv, mri, _ = lax.fori_loop(
                0, NS, m_body, (neg_inf, jnp.zeros((L,), jnp.int32), neg_inf))

            best = jnp.flip(mri, 0)
            tmpi_v[...] = best
            plsc.store_scatter(idx8_v, [lanes * 8], best)

            k_out = cid * (K // 2) + sid
            pltpu.async_copy(vals_hbm.at[idx8_v.at[pl.ds(k_out * 8, 1)]],
                             row_v, sem_row).wait()
            pltpu.sync_copy(row_v, out_hbm.at[pl.ds(k_out, 1)])


def kernel(query_hidden, keys, values, W_key, top_k):
    del top_k
    return _retrieve(query_hidden, keys.reshape(-1), values,
                     W_key.reshape(-1))
